# Initial kernel scaffold; baseline (speedup 1.0000x reference)
import jax
import jax.numpy as jnp
from jax.experimental import pallas as pl


def kernel(x, table, pos):
    raise NotImplementedError("write your pallas kernel here")


# speedup vs baseline: 1.3956x; 1.3956x over previous
import functools

import jax
import jax.numpy as jnp
from jax import lax
from jax.experimental import pallas as pl
from jax.experimental.pallas import tpu as pltpu
from jax.experimental.pallas import tpu_sc as plsc


@functools.lru_cache(maxsize=None)
def _build_gather(T: int, V: int, D: int):
    info = plsc.get_sparse_core_info()
    NC, NS = info.num_cores, info.num_subcores
    NW = NC * NS
    assert T % NW == 0, (T, NW)
    b_per_w = T // NW
    CHUNK = 32
    assert b_per_w % CHUNK == 0
    n_chunks = b_per_w // CHUNK

    mesh = plsc.VectorSubcoreMesh(core_axis_name="c", subcore_axis_name="s")

    @functools.partial(
        pl.kernel,
        mesh=mesh,
        out_type=jax.ShapeDtypeStruct((T, D), jnp.float32),
        scratch_types=[
            pltpu.VMEM((16,), jnp.int32),
            pltpu.VMEM((CHUNK, D), jnp.float32),
        ],
    )
    def gather_kernel(table_hbm, idx_hbm, out_hbm, idx_v, buf):
        wid = lax.axis_index("s") * NC + lax.axis_index("c")
        base = wid * b_per_w
        pltpu.sync_copy(idx_hbm.at[pl.ds(0, 16)], idx_v)
        pos0 = pl.multiple_of(idx_v[...][0], 8)
        for c in range(n_chunks):
            pltpu.sync_copy(
                table_hbm.at[pl.ds(pos0 + base + c * CHUNK, CHUNK)], buf)
            pltpu.sync_copy(
                buf, out_hbm.at[pl.ds(base + c * CHUNK, CHUNK)])

    return gather_kernel


def kernel(x, table, pos):
    T = x.shape[1]
    V, D = table.shape
    idx = jnp.arange(T, dtype=jnp.int32) + jnp.asarray(pos, dtype=jnp.int32)
    out = _build_gather(T, V, D)(table, idx)
    return out[None]
